# baseline (speedup 1.0000x reference)
import functools

import jax
import jax.numpy as jnp
from jax import lax
from jax.experimental import pallas as pl
from jax.experimental.pallas import tpu as pltpu
from jax.experimental.pallas import tpu_sc as plsc

_info = plsc.get_sparse_core_info()
_NC, _NS = _info.num_cores, _info.num_subcores
_NW = _NC * _NS
_NBUF = 4


def _sc_gather(idx_rs, table, nch, ch, d):
    mesh = plsc.VectorSubcoreMesh(core_axis_name="c", subcore_axis_name="s")

    @functools.partial(
        pl.kernel,
        mesh=mesh,
        out_type=jax.ShapeDtypeStruct((_NW, nch, ch, d), jnp.float32),
        scratch_types=(
            [pltpu.VMEM((nch, ch), jnp.int32)]
            + [pltpu.VMEM((ch, d), jnp.float32) for _ in range(_NBUF)]
            + [pltpu.SemaphoreType.DMA for _ in range(2 * _NBUF)]
        ),
    )
    def gather_kernel(idx_hbm, table_hbm, out_hbm, idx_v, *bufs_sems):
        bufs = bufs_sems[:_NBUF]
        gsems = bufs_sems[_NBUF:2 * _NBUF]
        ssems = bufs_sems[2 * _NBUF:]
        wid = lax.axis_index("s") * _NC + lax.axis_index("c")
        pltpu.sync_copy(idx_hbm.at[wid], idx_v)

        def start_gather(c):
            b = c % _NBUF
            return pltpu.async_copy(table_hbm.at[idx_v.at[c]], bufs[b],
                                    gsems[b])

        def start_store(c):
            b = c % _NBUF
            return pltpu.async_copy(bufs[b], out_hbm.at[wid, c], ssems[b])

        gd = [None] * nch
        sd = [None] * nch
        gd[0] = start_gather(0)
        gd[1] = start_gather(1)
        for c in range(nch):
            gd[c].wait()
            sd[c] = start_store(c)
            nc = c + 2
            if nc < nch:
                if nc - _NBUF >= 0:
                    sd[nc - _NBUF].wait()
                gd[nc] = start_gather(nc)
        for c in range(max(0, nch - _NBUF), nch):
            sd[c].wait()

    return gather_kernel(idx_rs, table)


def _tc_project(x, w, bias2d, n, d, e, bm):
    def mm(x_ref, w_ref, b_ref, o_ref):
        o_ref[...] = lax.dot_general(
            x_ref[...], w_ref[...],
            dimension_numbers=(((1,), (1,)), ((), ())),
            preferred_element_type=jnp.float32,
        ) + b_ref[...]

    return pl.pallas_call(
        mm,
        grid=(n // bm,),
        in_specs=[
            pl.BlockSpec((bm, d), lambda i: (i, 0)),
            pl.BlockSpec((e, d), lambda i: (0, 0)),
            pl.BlockSpec((1, e), lambda i: (0, 0)),
        ],
        out_specs=pl.BlockSpec((bm, e), lambda i: (i, 0)),
        out_shape=jax.ShapeDtypeStruct((n, e), jnp.float32),
    )(x, w, bias2d)


def kernel(idx, tok_emb_table, proj_w, proj_b):
    bsz, t = idx.shape
    v, d = tok_emb_table.shape
    e = proj_w.shape[0]
    n = bsz * t
    n_per_w = n // _NW
    ch = 32
    nch = n_per_w // ch

    k_chunks = 2
    nk = n // k_chunks
    nch_k = (nk // _NW) // ch
    idx_rs = idx.reshape(-1).astype(jnp.int32).reshape(k_chunks, _NW, nch_k,
                                                       ch)
    bias2d = proj_b.reshape(1, e)
    gs = [_sc_gather(idx_rs[k], tok_emb_table, nch_k, ch, d)
          for k in range(k_chunks)]
    ys = [_tc_project(g.reshape(nk, d), proj_w, bias2d, nk, d, e, bm=4096)
          for g in gs]
    y = jnp.concatenate(ys, axis=0)
    return y.reshape(bsz, t, e)


# speedup vs baseline: 1.3732x; 1.3732x over previous
import functools

import jax
import jax.numpy as jnp
from jax import lax
from jax.experimental import pallas as pl
from jax.experimental.pallas import tpu as pltpu
from jax.experimental.pallas import tpu_sc as plsc

_info = plsc.get_sparse_core_info()
_NC, _NS = _info.num_cores, _info.num_subcores
_NW = _NC * _NS
_NBUF = 4


def _sc_gather(idx_rs, table, nch, ch, d):
    mesh = plsc.VectorSubcoreMesh(core_axis_name="c", subcore_axis_name="s")

    @functools.partial(
        pl.kernel,
        mesh=mesh,
        out_type=jax.ShapeDtypeStruct((_NW, nch, ch, d), jnp.float32),
        scratch_types=(
            [pltpu.VMEM((nch, ch), jnp.int32)]
            + [pltpu.VMEM((ch, d), jnp.float32) for _ in range(_NBUF)]
            + [pltpu.SemaphoreType.DMA for _ in range(2 * _NBUF)]
        ),
    )
    def gather_kernel(idx_hbm, table_hbm, out_hbm, idx_v, *bufs_sems):
        bufs = bufs_sems[:_NBUF]
        gsems = bufs_sems[_NBUF:2 * _NBUF]
        ssems = bufs_sems[2 * _NBUF:]
        wid = lax.axis_index("s") * _NC + lax.axis_index("c")
        pltpu.sync_copy(idx_hbm.at[wid], idx_v)

        def start_gather(c):
            b = c % _NBUF
            return pltpu.async_copy(table_hbm.at[idx_v.at[c]], bufs[b],
                                    gsems[b])

        def start_store(c):
            b = c % _NBUF
            return pltpu.async_copy(bufs[b], out_hbm.at[wid, c], ssems[b])

        gd = [None] * nch
        sd = [None] * nch
        gd[0] = start_gather(0)
        gd[1] = start_gather(1)
        for c in range(nch):
            gd[c].wait()
            sd[c] = start_store(c)
            nc = c + 2
            if nc < nch:
                if nc - _NBUF >= 0:
                    sd[nc - _NBUF].wait()
                gd[nc] = start_gather(nc)
        for c in range(max(0, nch - _NBUF), nch):
            sd[c].wait()

    return gather_kernel(idx_rs, table)


def _tc_project(x, w, bias2d, n, d, e, bm):
    def mm(x_ref, w_ref, b_ref, o_ref):
        o_ref[...] = lax.dot_general(
            x_ref[...], w_ref[...],
            dimension_numbers=(((1,), (1,)), ((), ())),
            preferred_element_type=jnp.float32,
        ) + b_ref[...]

    return pl.pallas_call(
        mm,
        grid=(n // bm,),
        in_specs=[
            pl.BlockSpec((bm, d), lambda i: (i, 0)),
            pl.BlockSpec((e, d), lambda i: (0, 0)),
            pl.BlockSpec((1, e), lambda i: (0, 0)),
        ],
        out_specs=pl.BlockSpec((bm, e), lambda i: (i, 0)),
        out_shape=jax.ShapeDtypeStruct((n, e), jnp.float32),
    )(x, w, bias2d)


def kernel(idx, tok_emb_table, proj_w, proj_b):
    bsz, t = idx.shape
    v, d = tok_emb_table.shape
    e = proj_w.shape[0]
    n = bsz * t
    n_per_w = n // _NW
    ch = 32
    nch = n_per_w // ch

    idx_rs = idx.reshape(-1).astype(jnp.int32).reshape(_NW, nch, ch)
    gathered = _sc_gather(idx_rs, tok_emb_table, nch, ch, d)
    x = gathered.reshape(n, d)
    y = _tc_project(x, proj_w, proj_b.reshape(1, e), n, d, e, bm=4096)
    return y.reshape(bsz, t, e)
